# baseline (speedup 1.0000x reference)
import functools

import jax
import jax.numpy as jnp
import numpy as np
from jax import lax
from jax.experimental import pallas as pl
from jax.experimental.pallas import tpu as pltpu
from jax.experimental.pallas import tpu_sc as plsc

B, C, A, D = 1024, 50, 4, 64
VOCAB0, VOCAB1 = 100000, 1000
CA = C * A

NC, NS = 2, 16
NW = NC * NS
BPW = B // NW
NB = 4
NCHUNK = BPW // NB
EV = NB * C
EV_PAD = 224
HALF = 112


def _pos_encoding_np():
    pos = np.arange(C)[:, np.newaxis].astype(np.float32)
    i = np.arange(D)[np.newaxis, :].astype(np.float32)
    angle = pos / np.power(10000, 2.0 * (np.floor(i / 2.0)) / np.float32(D))
    angle[:, 0::2] = np.sin(angle[:, 0::2])
    angle[:, 1::2] = np.cos(angle[:, 1::2])
    return angle


_POS = _pos_encoding_np()

_E = np.arange(EV_PAD)
_RMAP = np.minimum(_E // C, NB - 1).astype(np.int32)
_WMAP = ((_E % C) * A).astype(np.int32)


def _sc_body(inp_hbm, ta_hbm, tr_hbm, pos_hbm, wmap_hbm, rmap_hbm, out_hbm,
             inp_v, idx0_v, idx1_v, st0_v, st1_v, buf_v, pos_v, wmap_v,
             rmap_v, sem):
    wid = lax.axis_index("s") * NC + lax.axis_index("c")
    pltpu.sync_copy(pos_hbm, pos_v)
    pltpu.sync_copy(wmap_hbm, wmap_v)
    pltpu.sync_copy(rmap_hbm, rmap_v)

    def chunk(k, carry):
        b0 = wid * BPW + k * NB
        pltpu.sync_copy(inp_hbm.at[pl.ds(b0, NB)], inp_v)

        for g in range(EV_PAD // 16):
            r0 = rmap_v[pl.ds(g * 16, 16)]
            w0 = wmap_v[pl.ds(g * 16, 16)]
            f0 = plsc.load_gather(inp_v, [r0, w0])
            f1 = plsc.load_gather(inp_v, [r0, w0 + 1])
            r, off = g // 7, (g % 7) * 16
            idx0_v[r, pl.ds(off, 16)] = f0.astype(jnp.int32)
            idx1_v[r, pl.ds(off, 16)] = f1.astype(jnp.int32)

        cps = []
        for j in range(2):
            cps.append(pltpu.async_copy(
                ta_hbm.at[idx0_v.at[j]], st0_v.at[pl.ds(j * HALF, HALF)], sem))
            cps.append(pltpu.async_copy(
                tr_hbm.at[idx1_v.at[j]], st1_v.at[pl.ds(j * HALF, HALF)], sem))
        for cp in cps:
            cp.wait()

        def ev_body(c, carry2):
            for b in range(NB):
                e = b * C + c
                rn = jnp.full((16,), b, jnp.int32)
                wn = jnp.full((16,), 2, jnp.int32) + c * A
                n0 = plsc.load_gather(inp_v, [rn, wn])
                n1 = plsc.load_gather(inp_v, [rn, wn + 1])
                for j in range(D // 16):
                    p = pos_v[c, pl.ds(j * 16, 16)]
                    v0 = st0_v[e, pl.ds(j * 16, 16)]
                    v1 = st1_v[e, pl.ds(j * 16, 16)]
                    buf_v[b, c * A, pl.ds(j * 16, 16)] = v0 + p
                    buf_v[b, c * A + 1, pl.ds(j * 16, 16)] = v1 + p
                    buf_v[b, c * A + 2, pl.ds(j * 16, 16)] = n0 + p
                    buf_v[b, c * A + 3, pl.ds(j * 16, 16)] = n1 + p
            return carry2

        lax.fori_loop(0, C, ev_body, 0)

        pltpu.sync_copy(buf_v, out_hbm.at[pl.ds(b0, NB)])
        return carry

    lax.fori_loop(0, NCHUNK, chunk, 0)


def kernel(inputs, table_activity, table_resource):
    pos = jnp.asarray(_POS)
    wmap = jnp.asarray(_WMAP)
    rmap = jnp.asarray(_RMAP)
    mesh = plsc.VectorSubcoreMesh(core_axis_name="c", subcore_axis_name="s")
    k = functools.partial(
        pl.kernel,
        out_type=jax.ShapeDtypeStruct((B, CA, D), jnp.float32),
        mesh=mesh,
        compiler_params=pltpu.CompilerParams(use_tc_tiling_on_sc=False,
                                             needs_layout_passes=False),
        scratch_types=[
            pltpu.VMEM((NB, CA), jnp.float32),
            pltpu.VMEM((2, HALF), jnp.int32),
            pltpu.VMEM((2, HALF), jnp.int32),
            pltpu.VMEM((EV_PAD, D), jnp.float32),
            pltpu.VMEM((EV_PAD, D), jnp.float32),
            pltpu.VMEM((NB, CA, D), jnp.float32),
            pltpu.VMEM((C, D), jnp.float32),
            pltpu.VMEM((EV_PAD,), jnp.int32),
            pltpu.VMEM((EV_PAD,), jnp.int32),
            pltpu.SemaphoreType.DMA,
        ],
    )(_sc_body)
    return k(inputs, table_activity, table_resource, pos, wmap, rmap)


# speedup vs baseline: 1.0204x; 1.0204x over previous
import functools

import jax
import jax.numpy as jnp
import numpy as np
from jax import lax
from jax.experimental import pallas as pl
from jax.experimental.pallas import tpu as pltpu
from jax.experimental.pallas import tpu_sc as plsc

B, C, A, D = 1024, 50, 4, 64
VOCAB0, VOCAB1 = 100000, 1000
CA = C * A

NC, NS = 2, 16
NW = NC * NS
BPW = B // NW
NB = 2
NCHUNK = BPW // NB
EV = NB * C


def _pos_encoding_np():
    pos = np.arange(C)[:, np.newaxis].astype(np.float32)
    i = np.arange(D)[np.newaxis, :].astype(np.float32)
    angle = pos / np.power(10000, 2.0 * (np.floor(i / 2.0)) / np.float32(D))
    angle[:, 0::2] = np.sin(angle[:, 0::2])
    angle[:, 1::2] = np.cos(angle[:, 1::2])
    return angle


_POS = _pos_encoding_np()


def _sc_body(idx0_hbm, idx1_hbm, num0_hbm, num1_hbm, ta_hbm, tr_hbm, pos_hbm,
             out_hbm, idx0_v, idx1_v, num0_v, num1_v, st0_v, st1_v, buf_v,
             pos_v, sem):
    wid = lax.axis_index("s") * NC + lax.axis_index("c")
    pltpu.sync_copy(pos_hbm, pos_v)

    def chunk(k, carry):
        b0 = wid * BPW + k * NB
        pltpu.sync_copy(idx0_hbm.at[pl.ds(b0, NB)], idx0_v)
        pltpu.sync_copy(idx1_hbm.at[pl.ds(b0, NB)], idx1_v)
        pltpu.sync_copy(num0_hbm.at[pl.ds(b0, NB)], num0_v)
        pltpu.sync_copy(num1_hbm.at[pl.ds(b0, NB)], num1_v)

        cps = []
        for b in range(NB):
            cps.append(pltpu.async_copy(
                ta_hbm.at[idx0_v.at[b]], st0_v.at[pl.ds(b * C, C)], sem))
            cps.append(pltpu.async_copy(
                tr_hbm.at[idx1_v.at[b]], st1_v.at[pl.ds(b * C, C)], sem))
        for cp in cps:
            cp.wait()

        def ev_body(c, carry2):
            for b in range(NB):
                e = b * C + c
                bsp = jnp.full((16,), b, jnp.int32)
                csp = jnp.full((16,), 0, jnp.int32) + c
                n0 = plsc.load_gather(num0_v, [bsp, csp])
                n1 = plsc.load_gather(num1_v, [bsp, csp])
                for j in range(D // 16):
                    p = pos_v[c, pl.ds(j * 16, 16)]
                    v0 = st0_v[e, pl.ds(j * 16, 16)]
                    v1 = st1_v[e, pl.ds(j * 16, 16)]
                    buf_v[b, c * A, pl.ds(j * 16, 16)] = v0 + p
                    buf_v[b, c * A + 1, pl.ds(j * 16, 16)] = v1 + p
                    buf_v[b, c * A + 2, pl.ds(j * 16, 16)] = n0 + p
                    buf_v[b, c * A + 3, pl.ds(j * 16, 16)] = n1 + p
            return carry2

        lax.fori_loop(0, C, ev_body, 0)

        pltpu.sync_copy(buf_v, out_hbm.at[pl.ds(b0, NB)])
        return carry

    lax.fori_loop(0, NCHUNK, chunk, 0)


def kernel(inputs, table_activity, table_resource):
    pos = jnp.asarray(_POS)
    idx0 = inputs[:, 0::4].astype(jnp.int32)
    idx1 = inputs[:, 1::4].astype(jnp.int32)
    num0 = inputs[:, 2::4]
    num1 = inputs[:, 3::4]
    ta128 = jnp.pad(table_activity, ((0, 0), (0, 128 - D)))
    tr128 = jnp.pad(table_resource, ((0, 0), (0, 128 - D)))
    mesh = plsc.VectorSubcoreMesh(core_axis_name="c", subcore_axis_name="s")
    k = functools.partial(
        pl.kernel,
        out_type=jax.ShapeDtypeStruct((B, CA, D), jnp.float32),
        mesh=mesh,
        compiler_params=pltpu.CompilerParams(use_tc_tiling_on_sc=True,
                                             needs_layout_passes=False),
        scratch_types=[
            pltpu.VMEM((NB, C), jnp.int32),
            pltpu.VMEM((NB, C), jnp.int32),
            pltpu.VMEM((NB, C), jnp.float32),
            pltpu.VMEM((NB, C), jnp.float32),
            pltpu.VMEM((EV, 128), jnp.float32),
            pltpu.VMEM((EV, 128), jnp.float32),
            pltpu.VMEM((NB, CA, D), jnp.float32),
            pltpu.VMEM((C, D), jnp.float32),
            pltpu.SemaphoreType.DMA,
        ],
    )(_sc_body)
    return k(idx0, idx1, num0, num1, ta128, tr128, pos)


# speedup vs baseline: 1.3685x; 1.3412x over previous
import functools

import jax
import jax.numpy as jnp
import numpy as np
from jax import lax
from jax.experimental import pallas as pl
from jax.experimental.pallas import tpu as pltpu
from jax.experimental.pallas import tpu_sc as plsc
from jax.experimental import layout as jex_layout

B, C, A, D = 1024, 50, 4, 64
VOCAB0, VOCAB1 = 100000, 1000
CA = C * A

NC, NS = 2, 16
NW = NC * NS
BPW = B // NW
NB = 2
NCHUNK = BPW // NB
EV = NB * C


def _pos_encoding_np():
    pos = np.arange(C)[:, np.newaxis].astype(np.float32)
    i = np.arange(D)[np.newaxis, :].astype(np.float32)
    angle = pos / np.power(10000, 2.0 * (np.floor(i / 2.0)) / np.float32(D))
    angle[:, 0::2] = np.sin(angle[:, 0::2])
    angle[:, 1::2] = np.cos(angle[:, 1::2])
    return angle


_POS = _pos_encoding_np()


def _sc_body(idx0_hbm, idx1_hbm, num0_hbm, num1_hbm, ta_hbm, tr_hbm, pos_hbm,
             out_hbm, idx0_v, idx1_v, num0_v, num1_v, st0_v, st1_v, buf_v,
             pos_v, sem):
    wid = lax.axis_index("s") * NC + lax.axis_index("c")
    pltpu.sync_copy(pos_hbm, pos_v)

    def chunk(k, carry):
        b0 = wid * BPW + k * NB
        pltpu.sync_copy(idx0_hbm.at[pl.ds(b0, NB)], idx0_v)
        pltpu.sync_copy(idx1_hbm.at[pl.ds(b0, NB)], idx1_v)
        pltpu.sync_copy(num0_hbm.at[pl.ds(b0, NB)], num0_v)
        pltpu.sync_copy(num1_hbm.at[pl.ds(b0, NB)], num1_v)

        cps = []
        for b in range(NB):
            cps.append(pltpu.async_copy(
                ta_hbm.at[idx0_v.at[b]], st0_v.at[pl.ds(b * C, C)], sem))
            cps.append(pltpu.async_copy(
                tr_hbm.at[idx1_v.at[b]], st1_v.at[pl.ds(b * C, C)], sem))
        for cp in cps:
            cp.wait()

        def ev_body(c, carry2):
            for b in range(NB):
                e = b * C + c
                bsp = jnp.full((16,), b, jnp.int32)
                csp = jnp.full((16,), 0, jnp.int32) + c
                n0 = plsc.load_gather(num0_v, [bsp, csp])
                n1 = plsc.load_gather(num1_v, [bsp, csp])
                for j in range(D // 16):
                    p = pos_v[c, pl.ds(j * 16, 16)]
                    v0 = st0_v[e, pl.ds(j * 16, 16)]
                    v1 = st1_v[e, pl.ds(j * 16, 16)]
                    buf_v[b, c * A, pl.ds(j * 16, 16)] = v0 + p
                    buf_v[b, c * A + 1, pl.ds(j * 16, 16)] = v1 + p
                    buf_v[b, c * A + 2, pl.ds(j * 16, 16)] = n0 + p
                    buf_v[b, c * A + 3, pl.ds(j * 16, 16)] = n1 + p
            return carry2

        lax.fori_loop(0, C, ev_body, 0)

        pltpu.sync_copy(buf_v, out_hbm.at[pl.ds(b0, NB)])
        return carry

    lax.fori_loop(0, NCHUNK, chunk, 0)


def kernel(inputs, table_activity, table_resource):
    pos = jnp.asarray(_POS)
    idx0 = inputs[:, 0::4].astype(jnp.int32)
    idx1 = inputs[:, 1::4].astype(jnp.int32)
    num0 = inputs[:, 2::4]
    num1 = inputs[:, 3::4]
    ta128 = jnp.pad(table_activity, ((0, 0), (0, 128 - D)))
    tr128 = jnp.pad(table_resource, ((0, 0), (0, 128 - D)))
    mesh = plsc.VectorSubcoreMesh(core_axis_name="c", subcore_axis_name="s")
    k = functools.partial(
        pl.kernel,
        out_type=jax.ShapeDtypeStruct((B, CA, D), jnp.float32),
        mesh=mesh,
        compiler_params=pltpu.CompilerParams(use_tc_tiling_on_sc=True,
                                             needs_layout_passes=False),
        scratch_types=[
            pltpu.VMEM((NB, C), jnp.int32),
            pltpu.VMEM((NB, C), jnp.int32),
            pltpu.VMEM((NB, C), jnp.float32),
            pltpu.VMEM((NB, C), jnp.float32),
            pltpu.VMEM((EV, 128), jnp.float32),
            pltpu.VMEM((EV, 128), jnp.float32),
            pltpu.VMEM((NB, CA, D), jnp.float32),
            pltpu.VMEM((C, D), jnp.float32),
            pltpu.SemaphoreType.DMA,
        ],
    )(_sc_body)
    out = k(idx0, idx1, num0, num1, ta128, tr128, pos)
    return jex_layout.with_layout_constraint(
        out, jex_layout.Layout(major_to_minor=(0, 1, 2)))
